# Initial kernel scaffold; baseline (speedup 1.0000x reference)
import jax
import jax.numpy as jnp
from jax.experimental import pallas as pl


def kernel(aff_idx, mat_idx, aff_table, mat_table, W, b):
    raise NotImplementedError("write your pallas kernel here")


# speedup vs baseline: 3.3499x; 3.3499x over previous
import functools

import jax
import jax.numpy as jnp
from jax import lax
from jax.experimental import pallas as pl
from jax.experimental.pallas import tpu as pltpu
from jax.experimental.pallas import tpu_sc as plsc

_AFF_DIM = 64
_OUT_DIM = 128
_N_AFF = 16
_N_MAT = 16


def _lut_body(aff_ref, mat_ref, wa_ref, wm_ref, b_ref, lut_ref):
    aff_proj = lax.dot_general(
        aff_ref[...], wa_ref[...], (((1,), (1,)), ((), ())),
        preferred_element_type=jnp.float32)
    mat_proj = lax.dot_general(
        mat_ref[...], wm_ref[...], (((1,), (1,)), ((), ())),
        preferred_element_type=jnp.float32)
    s = aff_proj[:, None, :] + mat_proj[None, :, :] + b_ref[...][None, :, :]
    lut_ref[...] = jnp.maximum(s, 0.0)


_lut_call = pl.pallas_call(
    _lut_body,
    out_shape=jax.ShapeDtypeStruct((_N_AFF, _N_MAT, _OUT_DIM), jnp.float32),
)

_NC = 2
_NS = 16
_NW = _NC * _NS
_B = 16384
_BPW = _B // _NW
_CH = 128
_NCH = _BPW // _CH

@functools.lru_cache(maxsize=1)
def _make_gather_kernel():
    mesh = plsc.VectorSubcoreMesh(core_axis_name="c", subcore_axis_name="s",
                                  num_cores=_NC, num_subcores=_NS)

    @functools.partial(
        pl.kernel,
        mesh=mesh,
        out_type=jax.ShapeDtypeStruct((_B, _OUT_DIM), jnp.float32),
        scratch_types=[
            pltpu.VMEM((_BPW,), jnp.int32),
            pltpu.VMEM((_BPW,), jnp.int32),
            pltpu.VMEM((_NCH, _CH), jnp.int32),
            pltpu.VMEM((_BPW, _OUT_DIM), jnp.float32),
            pltpu.SemaphoreType.DMA,
        ],
    )
    def gather_kernel(aff_hbm, mat_hbm, lut_hbm, out_hbm,
                      aidx_v, midx_v, cidx_v, rows_v, sem):
        wid = lax.axis_index("s") * _NC + lax.axis_index("c")
        base = wid * _BPW
        pltpu.sync_copy(aff_hbm.at[pl.ds(base, _BPW)], aidx_v)
        pltpu.sync_copy(mat_hbm.at[pl.ds(base, _BPW)], midx_v)
        for j in range(_NCH):
            for i in range(_CH // 16):
                src = pl.ds(j * _CH + i * 16, 16)
                cidx_v[j, pl.ds(i * 16, 16)] = aidx_v[src] * _N_MAT + midx_v[src]
        copies = [
            pltpu.async_copy(lut_hbm.at[cidx_v.at[j]],
                             rows_v.at[pl.ds(j * _CH, _CH)], sem)
            for j in range(_NCH)
        ]
        for c in copies:
            c.wait()
        pltpu.sync_copy(rows_v, out_hbm.at[pl.ds(base, _BPW)])

    return gather_kernel


def kernel(aff_idx, mat_idx, aff_table, mat_table, W, b):
    lut3 = _lut_call(aff_table, mat_table,
                     W[:, :_AFF_DIM], W[:, _AFF_DIM:], b.reshape(1, _OUT_DIM))
    lut = lut3.reshape(_N_AFF * _N_MAT, _OUT_DIM)
    return _make_gather_kernel()(aff_idx.astype(jnp.int32),
                                 mat_idx.astype(jnp.int32), lut)
